# Initial kernel scaffold; baseline (speedup 1.0000x reference)
import jax
import jax.numpy as jnp
from jax.experimental import pallas as pl


def kernel(x, W, C):
    raise NotImplementedError("write your pallas kernel here")


# speedup vs baseline: 2.7074x; 2.7074x over previous
import jax
import jax.numpy as jnp
from jax.experimental import pallas as pl
from jax.experimental.pallas import tpu as pltpu

D = 1024
K = 1024
LV = 4


def _rvq_body(x_ref, W_ref, C_ref, qout_ref, idx_ref, loss_ref, r_ref, q_ref):
    i = pl.program_id(0)

    @pl.when(i == 0)
    def _init():
        r_ref[...] = x_ref[...]
        q_ref[...] = jnp.zeros_like(q_ref)
        idx_ref[...] = jnp.zeros_like(idx_ref)
        loss_ref[...] = jnp.zeros_like(loss_ref)

    r = r_ref[...]
    W = W_ref[0]
    C = C_ref[0]

    p = jax.lax.dot_general(r, W, (((1,), (1,)), ((), ())),
                            preferred_element_type=jnp.float32)

    diff = p - C
    s = jnp.sum(diff * diff, axis=1, keepdims=True)
    idx = jnp.argmin(s)

    onehot = (jax.lax.broadcasted_iota(jnp.int32, (1, K), 1) == idx
              ).astype(jnp.float32)
    qrow = jax.lax.dot_general(onehot, C, (((1,), (0,)), ((), ())),
                               preferred_element_type=jnp.float32)

    q_new = q_ref[...] + qrow
    q_ref[...] = q_new
    r_ref[...] = r - qrow

    x = x_ref[...]
    dl = qrow - x
    loss_i = jnp.sum(dl * dl) / D

    lane = jax.lax.broadcasted_iota(jnp.int32, (1, 128), 1)
    idx_ref[...] = jnp.where(lane == i, idx, idx_ref[...])
    loss_ref[...] = jnp.where(lane == i, loss_i, loss_ref[...])

    @pl.when(i == LV - 1)
    def _final():
        qout_ref[...] = q_new
        dr = q_new - x
        recon = jnp.sum(dr * dr) / D
        loss_ref[...] = jnp.where(lane == LV, recon, loss_ref[...])


def kernel(x, W, C):
    qout, idx_pad, loss_pad = pl.pallas_call(
        _rvq_body,
        grid=(LV,),
        in_specs=[
            pl.BlockSpec((1, D), lambda i: (0, 0)),
            pl.BlockSpec((1, K, D), lambda i: (i, 0, 0)),
            pl.BlockSpec((1, K, D), lambda i: (i, 0, 0)),
        ],
        out_specs=[
            pl.BlockSpec((1, D), lambda i: (0, 0)),
            pl.BlockSpec((1, 128), lambda i: (0, 0)),
            pl.BlockSpec((1, 128), lambda i: (0, 0)),
        ],
        out_shape=[
            jax.ShapeDtypeStruct((1, D), jnp.float32),
            jax.ShapeDtypeStruct((1, 128), jnp.int32),
            jax.ShapeDtypeStruct((1, 128), jnp.float32),
        ],
        scratch_shapes=[
            pltpu.VMEM((1, D), jnp.float32),
            pltpu.VMEM((1, D), jnp.float32),
        ],
    )(x, W, C)
    return qout, idx_pad[0, :LV], loss_pad[0, :LV + 1]
